# Initial kernel scaffold; baseline (speedup 1.0000x reference)
import jax
import jax.numpy as jnp
from jax.experimental import pallas as pl


def kernel(x, edge_index, W_root1, W_neigh1, b1, W_root2, W_neigh2, b2):
    raise NotImplementedError("write your pallas kernel here")


# speedup vs baseline: 3.5093x; 3.5093x over previous
import functools

import jax
import jax.numpy as jnp
from jax import lax
from jax.experimental import pallas as pl
from jax.experimental.pallas import tpu as pltpu
from jax.experimental.pallas import tpu_sc as plsc

N = 10000
F = 128
H = 128
C = 64
E = 320000

N_PAD = 10240
NC = 2
NS = 16
NW = NC * NS
B = 128
NB = 80
EPT = B * NB
E_PAD = EPT * NW
RPT = N_PAD // NS


def _make_sc_agg(W, with_deg):
    mesh = plsc.VectorSubcoreMesh(core_axis_name="c", subcore_axis_name="s")
    out_type = [jax.ShapeDtypeStruct((N_PAD, W), jnp.float32),
                jax.ShapeDtypeStruct((N_PAD, W), jnp.float32)]
    if with_deg:
        out_type += [jax.ShapeDtypeStruct((N_PAD,), jnp.float32),
                     jax.ShapeDtypeStruct((N_PAD,), jnp.float32)]
    scratch = [
        pltpu.VMEM((NB, B), jnp.int32),
        pltpu.VMEM((NB, B), jnp.int32),
        pltpu.VMEM((B, W), jnp.float32),
        pltpu.VMEM_SHARED((N_PAD, W), jnp.float32),
        pltpu.SemaphoreType.DMA,
    ]
    if with_deg:
        scratch += [
            pltpu.VMEM((B,), jnp.float32),
            pltpu.VMEM_SHARED((N_PAD,), jnp.float32),
        ]

    def body(table, srcs, dsts, zrows, *rest):
        if with_deg:
            (zcol, ocol, out0, out1, deg0, deg1,
             src_v, dst_v, rows_v, acc_sh, sem, ones_v, deg_sh) = rest
        else:
            (out0, out1, src_v, dst_v, rows_v, acc_sh, sem) = rest
        cid = lax.axis_index("c")
        sid = lax.axis_index("s")
        wid = sid * NC + cid
        rows = pl.ds(sid * RPT, RPT)

        pltpu.sync_copy(zrows, acc_sh.at[rows])
        if with_deg:
            pltpu.sync_copy(zcol, deg_sh.at[rows])
            pltpu.sync_copy(ocol, ones_v)
        pltpu.sync_copy(srcs.at[wid], src_v)
        pltpu.sync_copy(dsts.at[wid], dst_v)
        plsc.subcore_barrier()

        def step(j, carry):
            pltpu.async_copy(table.at[src_v.at[j]], rows_v, sem).wait()
            pltpu.sync_copy(rows_v, acc_sh.at[dst_v.at[j]], add=True)
            if with_deg:
                pltpu.sync_copy(ones_v, deg_sh.at[dst_v.at[j]], add=True)
            return carry

        lax.fori_loop(0, NB, step, 0)
        plsc.subcore_barrier()

        @pl.when(cid == 0)
        def _():
            pltpu.sync_copy(acc_sh.at[rows], out0.at[rows])
            if with_deg:
                pltpu.sync_copy(deg_sh.at[rows], deg0.at[rows])

        @pl.when(cid == 1)
        def _():
            pltpu.sync_copy(acc_sh.at[rows], out1.at[rows])
            if with_deg:
                pltpu.sync_copy(deg_sh.at[rows], deg1.at[rows])

    return pl.kernel(body, out_type=out_type, mesh=mesh, scratch_types=scratch)


_sc_agg_l1 = _make_sc_agg(F, with_deg=True)
_sc_agg_l2 = _make_sc_agg(H, with_deg=False)

_R = 1024


def _dense1_body(agg0, agg1, deg0, deg1, x, wn1, wr1, b1, wr2, b2,
                 h_o, hr2_o, rdeg_o):
    rdeg = 1.0 / jnp.maximum(deg0[...] + deg1[...], 1.0)
    mean = (agg0[...] + agg1[...]) * rdeg
    h = jnp.maximum(
        jnp.dot(mean, wn1[...], preferred_element_type=jnp.float32)
        + jnp.dot(x[...], wr1[...], preferred_element_type=jnp.float32)
        + b1[...], 0.0)
    h_o[...] = h
    hr2_o[...] = (jnp.dot(h, wr2[...], preferred_element_type=jnp.float32)
                  + b2[...])
    rdeg_o[...] = rdeg


_dense1 = pl.pallas_call(
    _dense1_body,
    grid=(N_PAD // _R,),
    in_specs=[
        pl.BlockSpec((_R, H), lambda i: (i, 0)),
        pl.BlockSpec((_R, H), lambda i: (i, 0)),
        pl.BlockSpec((_R, 1), lambda i: (i, 0)),
        pl.BlockSpec((_R, 1), lambda i: (i, 0)),
        pl.BlockSpec((_R, F), lambda i: (i, 0)),
        pl.BlockSpec((F, H), lambda i: (0, 0)),
        pl.BlockSpec((F, H), lambda i: (0, 0)),
        pl.BlockSpec((1, H), lambda i: (0, 0)),
        pl.BlockSpec((H, C), lambda i: (0, 0)),
        pl.BlockSpec((1, C), lambda i: (0, 0)),
    ],
    out_specs=[
        pl.BlockSpec((_R, H), lambda i: (i, 0)),
        pl.BlockSpec((_R, C), lambda i: (i, 0)),
        pl.BlockSpec((_R, 1), lambda i: (i, 0)),
    ],
    out_shape=[
        jax.ShapeDtypeStruct((N_PAD, H), jnp.float32),
        jax.ShapeDtypeStruct((N_PAD, C), jnp.float32),
        jax.ShapeDtypeStruct((N_PAD, 1), jnp.float32),
    ],
)


def _dense2_body(agg0, agg1, rdeg, hr2, wn2, out_o):
    mean = (agg0[...] + agg1[...]) * rdeg[...]
    z = (jnp.dot(mean, wn2[...], preferred_element_type=jnp.float32)
         + hr2[...])
    m = jnp.max(z, axis=1, keepdims=True)
    zz = z - m
    out_o[...] = zz - jnp.log(jnp.sum(jnp.exp(zz), axis=1, keepdims=True))


_dense2 = pl.pallas_call(
    _dense2_body,
    grid=(N_PAD // _R,),
    in_specs=[
        pl.BlockSpec((_R, H), lambda i: (i, 0)),
        pl.BlockSpec((_R, H), lambda i: (i, 0)),
        pl.BlockSpec((_R, 1), lambda i: (i, 0)),
        pl.BlockSpec((_R, C), lambda i: (i, 0)),
        pl.BlockSpec((H, C), lambda i: (0, 0)),
    ],
    out_specs=pl.BlockSpec((_R, C), lambda i: (i, 0)),
    out_shape=jax.ShapeDtypeStruct((N_PAD, C), jnp.float32),
)


def kernel(x, edge_index, W_root1, W_neigh1, b1, W_root2, W_neigh2, b2):
    x_pad = jnp.pad(x, ((0, N_PAD - N), (0, 0)))
    src = edge_index[0]
    dst = edge_index[1]
    pad_e = E_PAD - E
    src_r = jnp.concatenate(
        [src, jnp.zeros((pad_e,), jnp.int32)]).reshape(NW, NB, B)
    dst_r = jnp.concatenate(
        [dst, jnp.full((pad_e,), N_PAD - 1, jnp.int32)]).reshape(NW, NB, B)
    zrows_f = jnp.zeros((RPT, F), jnp.float32)
    zcol = jnp.zeros((RPT,), jnp.float32)
    ocol = jnp.ones((B,), jnp.float32)

    agg0, agg1, deg0, deg1 = _sc_agg_l1(x_pad, src_r, dst_r,
                                        zrows_f, zcol, ocol)
    h, hr2, rdeg = _dense1(agg0, agg1,
                           deg0.reshape(N_PAD, 1), deg1.reshape(N_PAD, 1),
                           x_pad, W_neigh1, W_root1, b1.reshape(1, H),
                           W_root2, b2.reshape(1, C))
    a20, a21 = _sc_agg_l2(h, src_r, dst_r, zrows_f)
    out = _dense2(a20, a21, rdeg, hr2, W_neigh2)
    return out[:N]


# speedup vs baseline: 4.0004x; 1.1399x over previous
import functools

import jax
import jax.numpy as jnp
from jax import lax
from jax.experimental import pallas as pl
from jax.experimental.pallas import tpu as pltpu
from jax.experimental.pallas import tpu_sc as plsc

N = 10000
F = 128
H = 128
C = 64
E = 320000

N_PAD = 10240
NC = 2
NS = 16
NW = NC * NS
B = 128
NB = 80
EPT = B * NB
E_PAD = EPT * NW
RPT = N_PAD // NS
CH = 8
NCH = NB // CH


def _make_sc_agg(W, with_deg):
    mesh = plsc.VectorSubcoreMesh(core_axis_name="c", subcore_axis_name="s")
    out_type = [jax.ShapeDtypeStruct((N_PAD, W), jnp.float32),
                jax.ShapeDtypeStruct((N_PAD, W), jnp.float32)]
    if with_deg:
        out_type += [jax.ShapeDtypeStruct((N_PAD,), jnp.float32),
                     jax.ShapeDtypeStruct((N_PAD,), jnp.float32)]
    scratch = [
        pltpu.VMEM((2, CH, B), jnp.int32),
        pltpu.VMEM((2, CH, B), jnp.int32),
        pltpu.VMEM((2, B, W), jnp.float32),
        pltpu.VMEM_SHARED((N_PAD, W), jnp.float32),
        pltpu.SemaphoreType.DMA,
        pltpu.SemaphoreType.DMA,
        pltpu.SemaphoreType.DMA,
    ]
    if with_deg:
        scratch += [
            pltpu.VMEM((B,), jnp.float32),
            pltpu.VMEM_SHARED((N_PAD,), jnp.float32),
            pltpu.SemaphoreType.DMA,
        ]

    def body(table, srcs, dsts, zrows, *rest):
        if with_deg:
            (zcol, ocol, out0, out1, deg0, deg1,
             src_ch, dst_ch, rows_v, acc_sh, gsem, ssem, isem,
             ones_v, deg_sh, dsem) = rest
        else:
            (out0, out1, src_ch, dst_ch, rows_v, acc_sh,
             gsem, ssem, isem) = rest
        cid = lax.axis_index("c")
        sid = lax.axis_index("s")
        wid = sid * NC + cid
        rows = pl.ds(sid * RPT, RPT)

        def istage(c):
            cb = lax.rem(c, 2)
            sl = pl.ds(c * CH, CH)
            return (pltpu.make_async_copy(srcs.at[wid, sl],
                                          src_ch.at[cb], isem),
                    pltpu.make_async_copy(dsts.at[wid, sl],
                                          dst_ch.at[cb], isem))

        def gather(j):
            cb = lax.rem(lax.div(j, CH), 2)
            r = lax.rem(j, CH)
            return pltpu.make_async_copy(
                table.at[src_ch.at[cb, r]], rows_v.at[lax.rem(j, 2)], gsem)

        def scatter(j):
            cb = lax.rem(lax.div(j, CH), 2)
            r = lax.rem(j, CH)
            return pltpu.make_async_copy(
                rows_v.at[lax.rem(j, 2)], acc_sh.at[dst_ch.at[cb, r]], ssem)

        def dscat(j):
            cb = lax.rem(lax.div(j, CH), 2)
            r = lax.rem(j, CH)
            return pltpu.make_async_copy(
                ones_v, deg_sh.at[dst_ch.at[cb, r]], dsem)

        ia, ib = istage(0)
        ia.start()
        ib.start()
        pltpu.sync_copy(zrows, acc_sh.at[rows])
        if with_deg:
            pltpu.sync_copy(zcol, deg_sh.at[rows])
            pltpu.sync_copy(ocol, ones_v)
        plsc.subcore_barrier()
        iaw, ibw = istage(0)
        iaw.wait()
        ibw.wait()

        def step(j, carry):
            c = lax.div(j, CH)
            r = lax.rem(j, CH)

            @pl.when(jnp.logical_and(r == 0, c > 0))
            def _():
                wa, wb = istage(c)
                wa.wait()
                wb.wait()

            @pl.when(j >= 2)
            def _():
                scatter(j - 2).wait()

            gather(j).start()
            if with_deg:
                dscat(j).start(add=True)

            @pl.when(j >= 1)
            def _():
                gather(j - 1).wait()
                scatter(j - 1).start(add=True)
                if with_deg:
                    dscat(j - 1).wait()

            @pl.when(jnp.logical_and(r == 0, c + 1 < NCH))
            def _():
                na, nb_ = istage(c + 1)
                na.start()
                nb_.start()

            return carry

        lax.fori_loop(0, NB, step, 0)
        gather(NB - 1).wait()
        scatter(NB - 1).start(add=True)
        scatter(NB - 2).wait()
        scatter(NB - 1).wait()
        if with_deg:
            dscat(NB - 1).wait()
        plsc.subcore_barrier()

        @pl.when(cid == 0)
        def _():
            pltpu.sync_copy(acc_sh.at[rows], out0.at[rows])
            if with_deg:
                pltpu.sync_copy(deg_sh.at[rows], deg0.at[rows])

        @pl.when(cid == 1)
        def _():
            pltpu.sync_copy(acc_sh.at[rows], out1.at[rows])
            if with_deg:
                pltpu.sync_copy(deg_sh.at[rows], deg1.at[rows])

    return pl.kernel(body, out_type=out_type, mesh=mesh, scratch_types=scratch)


_sc_agg_l1 = _make_sc_agg(F, with_deg=True)
_sc_agg_l2 = _make_sc_agg(H, with_deg=False)

_R = 1024


def _dense1_body(agg0, agg1, deg0, deg1, x, wn1, wr1, b1, wr2, b2,
                 h_o, hr2_o, rdeg_o):
    rdeg = 1.0 / jnp.maximum(deg0[...] + deg1[...], 1.0)
    mean = (agg0[...] + agg1[...]) * rdeg
    h = jnp.maximum(
        jnp.dot(mean, wn1[...], preferred_element_type=jnp.float32)
        + jnp.dot(x[...], wr1[...], preferred_element_type=jnp.float32)
        + b1[...], 0.0)
    h_o[...] = h
    hr2_o[...] = (jnp.dot(h, wr2[...], preferred_element_type=jnp.float32)
                  + b2[...])
    rdeg_o[...] = rdeg


_dense1 = pl.pallas_call(
    _dense1_body,
    grid=(N_PAD // _R,),
    in_specs=[
        pl.BlockSpec((_R, H), lambda i: (i, 0)),
        pl.BlockSpec((_R, H), lambda i: (i, 0)),
        pl.BlockSpec((_R, 1), lambda i: (i, 0)),
        pl.BlockSpec((_R, 1), lambda i: (i, 0)),
        pl.BlockSpec((_R, F), lambda i: (i, 0)),
        pl.BlockSpec((F, H), lambda i: (0, 0)),
        pl.BlockSpec((F, H), lambda i: (0, 0)),
        pl.BlockSpec((1, H), lambda i: (0, 0)),
        pl.BlockSpec((H, C), lambda i: (0, 0)),
        pl.BlockSpec((1, C), lambda i: (0, 0)),
    ],
    out_specs=[
        pl.BlockSpec((_R, H), lambda i: (i, 0)),
        pl.BlockSpec((_R, C), lambda i: (i, 0)),
        pl.BlockSpec((_R, 1), lambda i: (i, 0)),
    ],
    out_shape=[
        jax.ShapeDtypeStruct((N_PAD, H), jnp.float32),
        jax.ShapeDtypeStruct((N_PAD, C), jnp.float32),
        jax.ShapeDtypeStruct((N_PAD, 1), jnp.float32),
    ],
)


def _dense2_body(agg0, agg1, rdeg, hr2, wn2, out_o):
    mean = (agg0[...] + agg1[...]) * rdeg[...]
    z = (jnp.dot(mean, wn2[...], preferred_element_type=jnp.float32)
         + hr2[...])
    m = jnp.max(z, axis=1, keepdims=True)
    zz = z - m
    out_o[...] = zz - jnp.log(jnp.sum(jnp.exp(zz), axis=1, keepdims=True))


_dense2 = pl.pallas_call(
    _dense2_body,
    grid=(N_PAD // _R,),
    in_specs=[
        pl.BlockSpec((_R, H), lambda i: (i, 0)),
        pl.BlockSpec((_R, H), lambda i: (i, 0)),
        pl.BlockSpec((_R, 1), lambda i: (i, 0)),
        pl.BlockSpec((_R, C), lambda i: (i, 0)),
        pl.BlockSpec((H, C), lambda i: (0, 0)),
    ],
    out_specs=pl.BlockSpec((_R, C), lambda i: (i, 0)),
    out_shape=jax.ShapeDtypeStruct((N_PAD, C), jnp.float32),
)


def kernel(x, edge_index, W_root1, W_neigh1, b1, W_root2, W_neigh2, b2):
    x_pad = jnp.pad(x, ((0, N_PAD - N), (0, 0)))
    src = edge_index[0]
    dst = edge_index[1]
    pad_e = E_PAD - E
    src_r = jnp.concatenate(
        [src, jnp.zeros((pad_e,), jnp.int32)]).reshape(NW, NB, B)
    dst_r = jnp.concatenate(
        [dst, jnp.full((pad_e,), N_PAD - 1, jnp.int32)]).reshape(NW, NB, B)
    zrows_f = jnp.zeros((RPT, F), jnp.float32)
    zcol = jnp.zeros((RPT,), jnp.float32)
    ocol = jnp.ones((B,), jnp.float32)

    agg0, agg1, deg0, deg1 = _sc_agg_l1(x_pad, src_r, dst_r,
                                        zrows_f, zcol, ocol)
    h, hr2, rdeg = _dense1(agg0, agg1,
                           deg0.reshape(N_PAD, 1), deg1.reshape(N_PAD, 1),
                           x_pad, W_neigh1, W_root1, b1.reshape(1, H),
                           W_root2, b2.reshape(1, C))
    a20, a21 = _sc_agg_l2(h, src_r, dst_r, zrows_f)
    out = _dense2(a20, a21, rdeg, hr2, W_neigh2)
    return out[:N]
